# baseline (speedup 1.0000x reference)
import functools

import jax
import jax.numpy as jnp
from jax import lax
from jax.experimental import pallas as pl
from jax.experimental.pallas import tpu as pltpu
from jax.experimental.pallas import tpu_sc as plsc

B = 4096
F = 26
V = 1000
D = 128
NV = D // 16
LPAD = 32
CH = 4
RPC = CH * F
LPC = CH * LPAD

_info = plsc.get_sparse_core_info()
NC, NS, L = _info.num_cores, _info.num_subcores, _info.num_lanes
NW = NC * NS
BPW = B // NW
NCH = BPW // CH
ROWS_PW = BPW * F

_mesh = plsc.VectorSubcoreMesh(core_axis_name="c", subcore_axis_name="s")


@functools.partial(
    pl.kernel,
    mesh=_mesh,
    out_type=jax.ShapeDtypeStruct((B,), jnp.float32),
    scratch_types=[
        pltpu.VMEM((RPC, D), jnp.float32),
        pltpu.VMEM((RPC, D), jnp.float32),
        pltpu.VMEM((LPC,), jnp.float32),
        pltpu.VMEM((LPC,), jnp.float32),
        pltpu.VMEM((RPC,), jnp.int32),
        pltpu.VMEM((RPC,), jnp.int32),
        pltpu.VMEM((LPC,), jnp.int32),
        pltpu.VMEM((LPC,), jnp.int32),
        pltpu.VMEM((16,), jnp.float32),
        pltpu.VMEM((BPW,), jnp.float32),
        pltpu.SemaphoreType.DMA,
        pltpu.SemaphoreType.DMA,
        pltpu.SemaphoreType.DMA,
        pltpu.SemaphoreType.DMA,
    ],
)
def _fcfm_sc(emb_hbm, eidx_hbm, lidx_hbm, lin_hbm, bias_hbm, out_hbm,
             rows0, rows1, lrow0, lrow1, eib0, eib1, lib0, lib1,
             bias_v, pre_v, sem_g0, sem_g1, sem_i0, sem_i1):
    rows = (rows0, rows1)
    lrow = (lrow0, lrow1)
    eib = (eib0, eib1)
    lib = (lib0, lib1)
    sem_g = (sem_g0, sem_g1)
    sem_i = (sem_i0, sem_i1)

    core = lax.axis_index("c")
    sub = lax.axis_index("s")
    w = core * NS + sub
    samp_base = w * BPW
    row_base = w * ROWS_PW
    lin_base = samp_base * LPAD

    pltpu.sync_copy(bias_hbm, bias_v)

    def start_idx(c, ib, lb, sem):
        pltpu.make_async_copy(
            eidx_hbm.at[pl.ds(row_base + c * RPC, RPC)], ib, sem).start()
        pltpu.make_async_copy(
            lidx_hbm.at[pl.ds(lin_base + c * LPC, LPC)], lb, sem).start()

    def wait_idx(c, ib, lb, sem):
        pltpu.make_async_copy(
            eidx_hbm.at[pl.ds(row_base + c * RPC, RPC)], ib, sem).wait()
        pltpu.make_async_copy(
            lidx_hbm.at[pl.ds(lin_base + c * LPC, LPC)], lb, sem).wait()

    def start_gather(rb, lb, ib, lb_i, sem):
        pltpu.make_async_copy(emb_hbm.at[ib], rb, sem).start()
        pltpu.make_async_copy(lin_hbm.at[lb_i], lb, sem).start()

    def wait_gather(rb, lb, ib, lb_i, sem):
        pltpu.make_async_copy(emb_hbm.at[ib], rb, sem).wait()
        pltpu.make_async_copy(lin_hbm.at[lb_i], lb, sem).wait()

    lanes = lax.iota(jnp.int32, L)
    zero = jnp.zeros((L,), jnp.float32)
    shuf = [(lanes + sh) & (L - 1) for sh in (8, 4, 2, 1)]

    def lane_sum(u):
        for idx in shuf:
            u = u + u.at[idx].get(mode="promise_in_bounds")
        return u

    def compute_chunk(c, buf, lbuf):
        contrib = zero
        for ss in range(CH):
            r0 = ss * F

            def fbody(f, acc):
                accs, accq = acc
                ns, nq = [], []
                for v in range(NV):
                    r = buf[r0 + f, pl.ds(v * L, L)]
                    ns.append(accs[v] + r)
                    nq.append(accq[v] + r * r)
                return (tuple(ns), tuple(nq))

            accs, accq = lax.fori_loop(
                0, F, fbody,
                (tuple(zero for _ in range(NV)),
                 tuple(zero for _ in range(NV))))
            t = accs[0] * accs[0] - accq[0]
            for v in range(1, NV):
                t = t + (accs[v] * accs[v] - accq[v])

            g0 = lbuf[pl.ds(ss * LPAD, L)]
            g1 = lbuf[pl.ds(ss * LPAD + L, L)]
            g1 = jnp.where(lanes < (F - L), g1, 0.0)

            pre = lane_sum(0.5 * t + g0 + g1)
            s_local = c * CH + ss
            contrib = contrib + jnp.where(
                lanes == lax.rem(s_local, L), pre, zero)

        win = (c // (L // CH)) * L
        plsc.addupdate(pre_v.at[pl.ds(win, L)], contrib)

    for v8 in range(BPW // L):
        pre_v[pl.ds(v8 * L, L)] = zero

    start_idx(0, eib[0], lib[0], sem_i[0])
    start_idx(1, eib[1], lib[1], sem_i[1])
    wait_idx(0, eib[0], lib[0], sem_i[0])
    start_gather(rows[0], lrow[0], eib[0], lib[0], sem_g[0])

    def ring_body(jj, _):
        for b in range(2):
            c = 2 * jj + b
            nb = 1 - b

            @pl.when(c + 1 < NCH)
            def _():
                wait_idx(c + 1, eib[nb], lib[nb], sem_i[nb])
                start_gather(rows[nb], lrow[nb], eib[nb], lib[nb], sem_g[nb])

            wait_gather(rows[b], lrow[b], eib[b], lib[b], sem_g[b])

            @pl.when(c + 2 < NCH)
            def _():
                start_idx(c + 2, eib[b], lib[b], sem_i[b])

            compute_chunk(c, rows[b], lrow[b])
        return 0

    lax.fori_loop(0, NCH // 2, ring_body, 0)

    bias_vec = bias_v[pl.ds(0, L)]
    for v8 in range(BPW // L):
        x = pre_v[pl.ds(v8 * L, L)] + bias_vec
        pre_v[pl.ds(v8 * L, L)] = 1.0 / (1.0 + jnp.exp(-x))

    pltpu.sync_copy(pre_v, out_hbm.at[pl.ds(samp_base, BPW)])


def kernel(indices, linear_tables, embed_tables, bias):
    idx32 = indices.astype(jnp.int32)
    gidx = idx32 + (jnp.arange(F, dtype=jnp.int32) * V)[None, :]
    eidx = gidx.reshape(B * F)
    lidx = jnp.concatenate(
        [gidx, jnp.zeros((B, LPAD - F), jnp.int32)], axis=1).reshape(B * LPAD)
    emb_flat = embed_tables.reshape(F * V, D)
    lin_flat = linear_tables.reshape(F * V)
    bias16 = jnp.broadcast_to(bias, (L,))
    out = _fcfm_sc(emb_flat, eidx, lidx, lin_flat, bias16)
    return out.reshape(B, 1)


# speedup vs baseline: 1.5161x; 1.5161x over previous
import functools

import jax
import jax.numpy as jnp
from jax import lax
from jax.experimental import pallas as pl
from jax.experimental.pallas import tpu as pltpu
from jax.experimental.pallas import tpu_sc as plsc

B = 4096
F = 26
V = 1000
D = 128
DW = 256
NV = D // 16
CH = 4
RPC = CH * F

_info = plsc.get_sparse_core_info()
NC, NS, L = _info.num_cores, _info.num_subcores, _info.num_lanes
NW = NC * NS
BPW = B // NW
NCH = BPW // CH
ROWS_PW = BPW * F

_mesh = plsc.VectorSubcoreMesh(core_axis_name="c", subcore_axis_name="s")


@functools.partial(
    pl.kernel,
    mesh=_mesh,
    out_type=jax.ShapeDtypeStruct((B,), jnp.float32),
    scratch_types=[
        pltpu.VMEM((RPC, DW), jnp.float32),
        pltpu.VMEM((RPC, DW), jnp.float32),
        pltpu.VMEM((RPC,), jnp.int32),
        pltpu.VMEM((RPC,), jnp.int32),
        pltpu.VMEM((16,), jnp.float32),
        pltpu.VMEM((BPW,), jnp.float32),
        pltpu.SemaphoreType.DMA,
        pltpu.SemaphoreType.DMA,
        pltpu.SemaphoreType.DMA,
        pltpu.SemaphoreType.DMA,
    ],
)
def _fcfm_sc(tab_hbm, eidx_hbm, bias_hbm, out_hbm,
             rows0, rows1, eib0, eib1,
             bias_v, pre_v, sem_g0, sem_g1, sem_i0, sem_i1):
    rows = (rows0, rows1)
    eib = (eib0, eib1)
    sem_g = (sem_g0, sem_g1)
    sem_i = (sem_i0, sem_i1)

    core = lax.axis_index("c")
    sub = lax.axis_index("s")
    w = core * NS + sub
    samp_base = w * BPW
    row_base = w * ROWS_PW

    pltpu.sync_copy(bias_hbm, bias_v)

    def start_idx(c, ib, sem):
        pltpu.make_async_copy(
            eidx_hbm.at[pl.ds(row_base + c * RPC, RPC)], ib, sem).start()

    def wait_idx(c, ib, sem):
        pltpu.make_async_copy(
            eidx_hbm.at[pl.ds(row_base + c * RPC, RPC)], ib, sem).wait()

    def start_gather(rb, ib, sem):
        pltpu.make_async_copy(tab_hbm.at[ib], rb, sem).start()

    def wait_gather(rb, ib, sem):
        pltpu.make_async_copy(tab_hbm.at[ib], rb, sem).wait()

    lanes = lax.iota(jnp.int32, L)
    zero = jnp.zeros((L,), jnp.float32)
    shuf = [(lanes + sh) & (L - 1) for sh in (8, 4, 2, 1)]

    def lane_sum(u):
        for idx in shuf:
            u = u + u.at[idx].get(mode="promise_in_bounds")
        return u

    def compute_chunk(c, buf):
        contrib = zero
        for ss in range(CH):
            r0 = ss * F

            def fbody(f, acc):
                accs, accq, lsum = acc
                ns, nq = [], []
                for v in range(NV):
                    r = buf[r0 + f, pl.ds(v * L, L)]
                    ns.append(accs[v] + r)
                    nq.append(accq[v] + r * r)
                lv = buf[r0 + f, pl.ds(NV * L, L)]
                return (tuple(ns), tuple(nq), lsum + lv)

            accs, accq, lsum = lax.fori_loop(
                0, F, fbody,
                (tuple(zero for _ in range(NV)),
                 tuple(zero for _ in range(NV)),
                 zero))
            t = accs[0] * accs[0] - accq[0]
            for v in range(1, NV):
                t = t + (accs[v] * accs[v] - accq[v])

            pre = lane_sum(0.5 * t + lsum)
            s_local = c * CH + ss
            contrib = contrib + jnp.where(
                lanes == lax.rem(s_local, L), pre, zero)

        win = (c // (L // CH)) * L
        plsc.addupdate(pre_v.at[pl.ds(win, L)], contrib)

    for v8 in range(BPW // L):
        pre_v[pl.ds(v8 * L, L)] = zero

    start_idx(0, eib[0], sem_i[0])
    start_idx(1, eib[1], sem_i[1])
    wait_idx(0, eib[0], sem_i[0])
    start_gather(rows[0], eib[0], sem_g[0])

    def ring_body(jj, _):
        for b in range(2):
            c = 2 * jj + b
            nb = 1 - b
            wait_gather(rows[b], eib[b], sem_g[b])

            @pl.when(c + 2 < NCH)
            def _():
                start_idx(c + 2, eib[b], sem_i[b])

            @pl.when(c + 1 < NCH)
            def _():
                wait_idx(c + 1, eib[nb], sem_i[nb])
                start_gather(rows[nb], eib[nb], sem_g[nb])

            compute_chunk(c, rows[b])
        return 0

    lax.fori_loop(0, NCH // 2, ring_body, 0)

    bias_vec = bias_v[pl.ds(0, L)]
    for v8 in range(BPW // L):
        x = pre_v[pl.ds(v8 * L, L)] + bias_vec
        pre_v[pl.ds(v8 * L, L)] = 1.0 / (1.0 + jnp.exp(-x))

    pltpu.sync_copy(pre_v, out_hbm.at[pl.ds(samp_base, BPW)])


def kernel(indices, linear_tables, embed_tables, bias):
    idx32 = indices.astype(jnp.int32)
    gidx = idx32 + (jnp.arange(F, dtype=jnp.int32) * V)[None, :]
    eidx = gidx.reshape(B * F)
    tab = jnp.concatenate(
        [embed_tables.reshape(F * V, D),
         linear_tables.reshape(F * V, 1),
         jnp.zeros((F * V, DW - D - 1), jnp.float32)], axis=1)
    bias16 = jnp.broadcast_to(bias, (L,))
    out = _fcfm_sc(tab, eidx, bias16)
    return out.reshape(B, 1)
